# Initial kernel scaffold; baseline (speedup 1.0000x reference)
import jax
import jax.numpy as jnp
from jax.experimental import pallas as pl


def kernel(x, edge_index, edge_attr, batch, vn0, wrel0, brel0, wroot0, gam0, bet0, vn1, wrel1, brel1, wroot1, gam1, bet1, vn2, wrel2, brel2, wroot2, gam2, bet2, wrel_p, brel_p, wroot_p, wmlp, bmlp):
    raise NotImplementedError("write your pallas kernel here")


# speedup vs baseline: 5.7376x; 5.7376x over previous
import functools

import jax
import jax.numpy as jnp
from jax import lax
from jax.experimental import pallas as pl
from jax.experimental.pallas import tpu as pltpu
from jax.experimental.pallas import tpu_sc as plsc

_N = 10000
_E = 320000
_H = 128
_OUT = 20
_EPS = 1e-5
_NC = 2
_NS = 16
_NW = _NC * _NS
_EPW = _E // _NW
_CH = 80
_NCHUNK = _EPW // _CH
_NP = 10240
_RPT = _NP // _NS


def _row_segsum_body(h_hbm, src_hbm, dst_hbm, z_hbm, out_hbm,
                     src_v, dst_v, rows_v, acc_sh, sem):
    c = lax.axis_index("c")
    s = lax.axis_index("s")
    wid = s * _NC + c
    row0 = s * _RPT
    pltpu.sync_copy(z_hbm, acc_sh.at[pl.ds(row0, _RPT)])
    plsc.subcore_barrier()
    base = wid * _EPW

    def body(j, carry):
        off = pl.multiple_of(base + j * _CH, 8)
        pltpu.sync_copy(src_hbm.at[pl.ds(off, _CH)], src_v)
        pltpu.async_copy(h_hbm.at[src_v], rows_v, sem).wait()
        pltpu.sync_copy(dst_hbm.at[pl.ds(off, _CH)], dst_v)
        pltpu.sync_copy(rows_v, acc_sh.at[dst_v], add=True)
        return carry

    lax.fori_loop(0, _NCHUNK, body, 0)
    plsc.subcore_barrier()
    pltpu.sync_copy(acc_sh.at[pl.ds(row0, _RPT)], out_hbm.at[c, pl.ds(row0, _RPT)])


def _sc_row_segsum(h, src, dst, ztile):
    mesh = plsc.VectorSubcoreMesh(core_axis_name="c", subcore_axis_name="s")
    kern = pl.kernel(
        _row_segsum_body,
        out_type=jax.ShapeDtypeStruct((_NC, _NP, _H), jnp.float32),
        mesh=mesh,
        scratch_types=[
            pltpu.VMEM((_CH,), jnp.int32),
            pltpu.VMEM((_CH,), jnp.int32),
            pltpu.VMEM((_CH, _H), jnp.float32),
            pltpu.VMEM_SHARED((_NP, _H), jnp.float32),
            pltpu.SemaphoreType.DMA,
        ],
    )
    return kern(h, src, dst, ztile)


def _scal_segsum_body(srel_hbm, src_hbm, dst_hbm, z_hbm, out_hbm,
                      srel_v, src_v, dst_v, acc_v):
    c = lax.axis_index("c")
    s = lax.axis_index("s")
    wid = s * _NC + c
    pltpu.sync_copy(srel_hbm, srel_v)
    pltpu.sync_copy(z_hbm, acc_v)
    base = wid * _EPW
    pltpu.sync_copy(src_hbm.at[pl.ds(base, _EPW)], src_v)
    pltpu.sync_copy(dst_hbm.at[pl.ds(base, _EPW)], dst_v)

    def body(i, carry):
        off = pl.multiple_of(i * 16, 16)
        idx = src_v[pl.ds(off, 16)]
        vals = plsc.load_gather(srel_v, [idx])
        didx = dst_v[pl.ds(off, 16)]
        plsc.addupdate_scatter(acc_v, [didx], vals)
        return carry

    lax.fori_loop(0, _EPW // 16, body, 0)
    pltpu.sync_copy(acc_v, out_hbm.at[pl.ds(wid * _N, _N)])


def _sc_scal_segsum(srel, src, dst, zvec):
    mesh = plsc.VectorSubcoreMesh(core_axis_name="c", subcore_axis_name="s")
    kern = pl.kernel(
        _scal_segsum_body,
        out_type=jax.ShapeDtypeStruct((_NW * _N,), jnp.float32),
        mesh=mesh,
        compiler_params=pltpu.CompilerParams(needs_layout_passes=False),
        scratch_types=[
            pltpu.VMEM((_N,), jnp.float32),
            pltpu.VMEM((_EPW,), jnp.int32),
            pltpu.VMEM((_EPW,), jnp.int32),
            pltpu.VMEM((_N,), jnp.float32),
        ],
    )
    return kern(srel, src, dst, zvec)


def _pre_body(x_ref, vn_ref, o_ref):
    o_ref[...] = x_ref[...] + vn_ref[...]


def _tc_pre(x, vn0):
    return pl.pallas_call(
        _pre_body,
        out_shape=jax.ShapeDtypeStruct((_N, _H), jnp.float32),
    )(x, vn0)


def _dense_body(p_ref, h_ref, wr_ref, br_ref, ws_ref, g_ref, b_ref,
                vn_ref, o_ref):
    agg = p_ref[0, pl.ds(0, _N), :] + p_ref[1, pl.ds(0, _N), :]
    h = h_ref[...]
    z = (jnp.dot(agg, wr_ref[...], preferred_element_type=jnp.float32)
         + br_ref[...]
         + jnp.dot(h, ws_ref[...], preferred_element_type=jnp.float32))
    mean = jnp.mean(z, axis=0, keepdims=True)
    zc = z - mean
    var = jnp.mean(zc * zc, axis=0, keepdims=True)
    hn = g_ref[...] * zc * lax.rsqrt(var + _EPS) + b_ref[...]
    o_ref[...] = jnp.maximum(hn, 0.0) + vn_ref[...]


def _dense_last_body(p_ref, h_ref, wr_ref, br_ref, ws_ref, g_ref, b_ref,
                     wrp_ref, wsp_ref, o_ref, srel_ref, sroot_ref):
    agg = p_ref[0, pl.ds(0, _N), :] + p_ref[1, pl.ds(0, _N), :]
    h = h_ref[...]
    z = (jnp.dot(agg, wr_ref[...], preferred_element_type=jnp.float32)
         + br_ref[...]
         + jnp.dot(h, ws_ref[...], preferred_element_type=jnp.float32))
    mean = jnp.mean(z, axis=0, keepdims=True)
    zc = z - mean
    var = jnp.mean(zc * zc, axis=0, keepdims=True)
    hn = g_ref[...] * zc * lax.rsqrt(var + _EPS) + b_ref[...]
    h3 = jnp.maximum(hn, 0.0)
    o_ref[...] = h3
    srel_ref[...] = jnp.dot(h3, wrp_ref[...], preferred_element_type=jnp.float32)
    sroot_ref[...] = jnp.dot(h3, wsp_ref[...], preferred_element_type=jnp.float32)


def _tc_dense(p, h, wr, br, ws, g, b, vn_next):
    return pl.pallas_call(
        _dense_body,
        out_shape=jax.ShapeDtypeStruct((_N, _H), jnp.float32),
    )(p, h, wr, br, ws, g, b, vn_next)


def _tc_dense_last(p, h, wr, br, ws, g, b, wrp, wsp):
    return pl.pallas_call(
        _dense_last_body,
        out_shape=(
            jax.ShapeDtypeStruct((_N, _H), jnp.float32),
            jax.ShapeDtypeStruct((_N, 1), jnp.float32),
            jax.ShapeDtypeStruct((_N, 1), jnp.float32),
        ),
    )(p, h, wr, br, ws, g, b, wrp, wsp)


def _final_body(parts_ref, sroot_ref, brp_ref, h3_ref, wmlp_ref, bmlp_ref,
                o_ref):
    score = (jnp.sum(parts_ref[...], axis=0, keepdims=True)
             + sroot_ref[...] + brp_ref[...])
    m = jnp.max(score)
    iota = lax.broadcasted_iota(jnp.int32, (1, _N), 1)
    am = jnp.min(jnp.where(score == m, iota, _N))
    row = h3_ref[pl.ds(am, 1), :]
    hp = row * jnp.tanh(m)
    o_ref[...] = (jnp.dot(hp, wmlp_ref[...], preferred_element_type=jnp.float32)
                  + bmlp_ref[...])


def _tc_final(parts, sroot, brp, h3, wmlp, bmlp):
    return pl.pallas_call(
        _final_body,
        out_shape=jax.ShapeDtypeStruct((1, _OUT), jnp.float32),
    )(parts, sroot, brp, h3, wmlp, bmlp)


def kernel(x, edge_index, edge_attr, batch, vn0, wrel0, brel0, wroot0, gam0,
           bet0, vn1, wrel1, brel1, wroot1, gam1, bet1, vn2, wrel2, brel2,
           wroot2, gam2, bet2, wrel_p, brel_p, wroot_p, wmlp, bmlp):
    del edge_attr, batch
    src = edge_index[0]
    dst = edge_index[1]
    ztile = jnp.zeros((_RPT, _H), jnp.float32)
    zvec = jnp.zeros((_N,), jnp.float32)

    layers = [
        (wrel0, brel0.reshape(1, _H), wroot0, gam0.reshape(1, _H),
         bet0.reshape(1, _H)),
        (wrel1, brel1.reshape(1, _H), wroot1, gam1.reshape(1, _H),
         bet1.reshape(1, _H)),
        (wrel2, brel2.reshape(1, _H), wroot2, gam2.reshape(1, _H),
         bet2.reshape(1, _H)),
    ]

    h = _tc_pre(x, vn0)
    for i in range(2):
        wr, br, ws, g, b = layers[i]
        p = _sc_row_segsum(h, src, dst, ztile)
        vn_next = vn1 if i == 0 else vn2
        h = _tc_dense(p, h, wr, br, ws, g, b, vn_next)

    wr, br, ws, g, b = layers[2]
    p = _sc_row_segsum(h, src, dst, ztile)
    h3, srel, sroot = _tc_dense_last(p, h, wr, br, ws, g, b, wrel_p, wroot_p)

    parts = _sc_scal_segsum(srel.reshape(_N), src, dst, zvec).reshape(_NW, _N)
    out = _tc_final(parts, sroot.reshape(1, _N), brel_p.reshape(1, 1), h3,
                    wmlp, bmlp)
    return out


# speedup vs baseline: 10.9336x; 1.9056x over previous
import functools

import jax
import jax.numpy as jnp
from jax import lax
from jax.experimental import pallas as pl
from jax.experimental.pallas import tpu as pltpu
from jax.experimental.pallas import tpu_sc as plsc

_N = 10000
_E = 320000
_H = 128
_OUT = 20
_EPS = 1e-5
_NC = 2
_NS = 16
_NW = _NC * _NS
_EPW = _E // _NW
_CH = 128
_NCHUNK = 80
_EPWP = _NCHUNK * _CH
_EPAD = _NW * _EPWP - _E
_NHALF = _NCHUNK // 2
_NP = 10240
_RPT = _NP // _NS


def _row_segsum_body(h_hbm, src_hbm, dst_hbm, z_hbm, out_hbm,
                     src_v, dstb0, dstb1, buf0, buf1, acc_sh,
                     sem_g0, sem_g1, sem_s0, sem_s1, sem_d0, sem_d1):
    c = lax.axis_index("c")
    s = lax.axis_index("s")
    wid = s * _NC + c
    row0 = s * _RPT
    pltpu.sync_copy(z_hbm, acc_sh.at[pl.ds(row0, _RPT)])
    pltpu.sync_copy(src_hbm.at[wid], src_v)
    plsc.subcore_barrier()

    pltpu.async_copy(dst_hbm.at[wid, 0], dstb0, sem_d0)
    pltpu.async_copy(h_hbm.at[src_v.at[0]], buf0, sem_g0)

    def body(t, carry):
        j0 = t * 2
        j1 = j0 + 1

        @pl.when(t > 0)
        def _():
            pltpu.make_async_copy(buf1, acc_sh.at[dstb1], sem_s1).wait()

        pltpu.async_copy(dst_hbm.at[wid, j1], dstb1, sem_d1)
        pltpu.async_copy(h_hbm.at[src_v.at[j1]], buf1, sem_g1)
        pltpu.make_async_copy(h_hbm.at[src_v.at[j0]], buf0, sem_g0).wait()
        pltpu.make_async_copy(dst_hbm.at[wid, j0], dstb0, sem_d0).wait()
        pltpu.async_copy(buf0, acc_sh.at[dstb0], sem_s0, add=True)
        pltpu.make_async_copy(h_hbm.at[src_v.at[j1]], buf1, sem_g1).wait()
        pltpu.make_async_copy(dst_hbm.at[wid, j1], dstb1, sem_d1).wait()
        pltpu.async_copy(buf1, acc_sh.at[dstb1], sem_s1, add=True)
        pltpu.make_async_copy(buf0, acc_sh.at[dstb0], sem_s0).wait()

        @pl.when(t < _NHALF - 1)
        def _():
            pltpu.async_copy(dst_hbm.at[wid, j0 + 2], dstb0, sem_d0)
            pltpu.async_copy(h_hbm.at[src_v.at[j0 + 2]], buf0, sem_g0)

        return carry

    lax.fori_loop(0, _NHALF, body, 0)
    pltpu.make_async_copy(buf1, acc_sh.at[dstb1], sem_s1).wait()
    plsc.subcore_barrier()
    pltpu.sync_copy(acc_sh.at[pl.ds(row0, _RPT)], out_hbm.at[c, pl.ds(row0, _RPT)])


def _sc_row_segsum(h, src2, dst2, ztile):
    mesh = plsc.VectorSubcoreMesh(core_axis_name="c", subcore_axis_name="s")
    kern = pl.kernel(
        _row_segsum_body,
        out_type=jax.ShapeDtypeStruct((_NC, _NP, _H), jnp.float32),
        mesh=mesh,
        scratch_types=[
            pltpu.VMEM((_NCHUNK, _CH), jnp.int32),
            pltpu.VMEM((_CH,), jnp.int32),
            pltpu.VMEM((_CH,), jnp.int32),
            pltpu.VMEM((_CH, _H), jnp.float32),
            pltpu.VMEM((_CH, _H), jnp.float32),
            pltpu.VMEM_SHARED((_NP, _H), jnp.float32),
            pltpu.SemaphoreType.DMA,
            pltpu.SemaphoreType.DMA,
            pltpu.SemaphoreType.DMA,
            pltpu.SemaphoreType.DMA,
            pltpu.SemaphoreType.DMA,
            pltpu.SemaphoreType.DMA,
        ],
    )
    return kern(h, src2, dst2, ztile)


def _scal_segsum_body(srel_hbm, src_hbm, dst_hbm, z_hbm, out_hbm,
                      srel_v, src_v, dst_v, acc_v):
    c = lax.axis_index("c")
    s = lax.axis_index("s")
    wid = s * _NC + c
    pltpu.sync_copy(srel_hbm, srel_v)
    pltpu.sync_copy(z_hbm, acc_v)
    base = wid * _EPW
    pltpu.sync_copy(src_hbm.at[pl.ds(base, _EPW)], src_v)
    pltpu.sync_copy(dst_hbm.at[pl.ds(base, _EPW)], dst_v)

    def body(i, carry):
        off = pl.multiple_of(i * 16, 16)
        idx = src_v[pl.ds(off, 16)]
        vals = plsc.load_gather(srel_v, [idx])
        didx = dst_v[pl.ds(off, 16)]
        plsc.addupdate_scatter(acc_v, [didx], vals)
        return carry

    lax.fori_loop(0, _EPW // 16, body, 0)
    pltpu.sync_copy(acc_v, out_hbm.at[pl.ds(wid * _N, _N)])


def _sc_scal_segsum(srel, src, dst, zvec):
    mesh = plsc.VectorSubcoreMesh(core_axis_name="c", subcore_axis_name="s")
    kern = pl.kernel(
        _scal_segsum_body,
        out_type=jax.ShapeDtypeStruct((_NW * _N,), jnp.float32),
        mesh=mesh,
        compiler_params=pltpu.CompilerParams(needs_layout_passes=False),
        scratch_types=[
            pltpu.VMEM((_N,), jnp.float32),
            pltpu.VMEM((_EPW,), jnp.int32),
            pltpu.VMEM((_EPW,), jnp.int32),
            pltpu.VMEM((_N,), jnp.float32),
        ],
    )
    return kern(srel, src, dst, zvec)


def _pre_body(x_ref, vn_ref, o_ref):
    o_ref[...] = x_ref[...] + vn_ref[...]


def _tc_pre(x, vn0):
    return pl.pallas_call(
        _pre_body,
        out_shape=jax.ShapeDtypeStruct((_N, _H), jnp.float32),
    )(x, vn0)


def _dense_body(p_ref, h_ref, wr_ref, br_ref, ws_ref, g_ref, b_ref,
                vn_ref, o_ref):
    agg = p_ref[0, pl.ds(0, _N), :] + p_ref[1, pl.ds(0, _N), :]
    h = h_ref[...]
    z = (jnp.dot(agg, wr_ref[...], preferred_element_type=jnp.float32)
         + br_ref[...]
         + jnp.dot(h, ws_ref[...], preferred_element_type=jnp.float32))
    mean = jnp.mean(z, axis=0, keepdims=True)
    zc = z - mean
    var = jnp.mean(zc * zc, axis=0, keepdims=True)
    hn = g_ref[...] * zc * lax.rsqrt(var + _EPS) + b_ref[...]
    o_ref[...] = jnp.maximum(hn, 0.0) + vn_ref[...]


def _dense_last_body(p_ref, h_ref, wr_ref, br_ref, ws_ref, g_ref, b_ref,
                     wrp_ref, wsp_ref, o_ref, srel_ref, sroot_ref):
    agg = p_ref[0, pl.ds(0, _N), :] + p_ref[1, pl.ds(0, _N), :]
    h = h_ref[...]
    z = (jnp.dot(agg, wr_ref[...], preferred_element_type=jnp.float32)
         + br_ref[...]
         + jnp.dot(h, ws_ref[...], preferred_element_type=jnp.float32))
    mean = jnp.mean(z, axis=0, keepdims=True)
    zc = z - mean
    var = jnp.mean(zc * zc, axis=0, keepdims=True)
    hn = g_ref[...] * zc * lax.rsqrt(var + _EPS) + b_ref[...]
    h3 = jnp.maximum(hn, 0.0)
    o_ref[...] = h3
    srel_ref[...] = jnp.dot(h3, wrp_ref[...], preferred_element_type=jnp.float32)
    sroot_ref[...] = jnp.dot(h3, wsp_ref[...], preferred_element_type=jnp.float32)


def _tc_dense(p, h, wr, br, ws, g, b, vn_next):
    return pl.pallas_call(
        _dense_body,
        out_shape=jax.ShapeDtypeStruct((_N, _H), jnp.float32),
    )(p, h, wr, br, ws, g, b, vn_next)


def _tc_dense_last(p, h, wr, br, ws, g, b, wrp, wsp):
    return pl.pallas_call(
        _dense_last_body,
        out_shape=(
            jax.ShapeDtypeStruct((_N, _H), jnp.float32),
            jax.ShapeDtypeStruct((_N, 1), jnp.float32),
            jax.ShapeDtypeStruct((_N, 1), jnp.float32),
        ),
    )(p, h, wr, br, ws, g, b, wrp, wsp)


def _final_body(parts_ref, sroot_ref, brp_ref, h3_ref, wmlp_ref, bmlp_ref,
                o_ref):
    score = (jnp.sum(parts_ref[...], axis=0, keepdims=True)
             + sroot_ref[...] + brp_ref[...])
    m = jnp.max(score)
    iota = lax.broadcasted_iota(jnp.int32, (1, _N), 1)
    am = jnp.min(jnp.where(score == m, iota, _N))
    row = h3_ref[pl.ds(am, 1), :]
    hp = row * jnp.tanh(m)
    o_ref[...] = (jnp.dot(hp, wmlp_ref[...], preferred_element_type=jnp.float32)
                  + bmlp_ref[...])


def _tc_final(parts, sroot, brp, h3, wmlp, bmlp):
    return pl.pallas_call(
        _final_body,
        out_shape=jax.ShapeDtypeStruct((1, _OUT), jnp.float32),
    )(parts, sroot, brp, h3, wmlp, bmlp)


def kernel(x, edge_index, edge_attr, batch, vn0, wrel0, brel0, wroot0, gam0,
           bet0, vn1, wrel1, brel1, wroot1, gam1, bet1, vn2, wrel2, brel2,
           wroot2, gam2, bet2, wrel_p, brel_p, wroot_p, wmlp, bmlp):
    del edge_attr, batch
    src = edge_index[0]
    dst = edge_index[1]
    fill = jnp.arange(_EPAD, dtype=jnp.int32)
    src_p = jnp.concatenate([src, (fill * 97) % _N])
    dst_p = jnp.concatenate([dst, _N + fill % (_NP - _N)])
    src2 = src_p.reshape(_NW, _NCHUNK, _CH)
    dst2 = dst_p.reshape(_NW, _NCHUNK, _CH)
    ztile = jnp.zeros((_RPT, _H), jnp.float32)
    zvec = jnp.zeros((_N,), jnp.float32)

    layers = [
        (wrel0, brel0.reshape(1, _H), wroot0, gam0.reshape(1, _H),
         bet0.reshape(1, _H)),
        (wrel1, brel1.reshape(1, _H), wroot1, gam1.reshape(1, _H),
         bet1.reshape(1, _H)),
        (wrel2, brel2.reshape(1, _H), wroot2, gam2.reshape(1, _H),
         bet2.reshape(1, _H)),
    ]

    h = _tc_pre(x, vn0)
    for i in range(2):
        wr, br, ws, g, b = layers[i]
        p = _sc_row_segsum(h, src2, dst2, ztile)
        vn_next = vn1 if i == 0 else vn2
        h = _tc_dense(p, h, wr, br, ws, g, b, vn_next)

    wr, br, ws, g, b = layers[2]
    p = _sc_row_segsum(h, src2, dst2, ztile)
    h3, srel, sroot = _tc_dense_last(p, h, wr, br, ws, g, b, wrel_p, wroot_p)

    parts = _sc_scal_segsum(srel.reshape(_N), src, dst, zvec).reshape(_NW, _N)
    out = _tc_final(parts, sroot.reshape(1, _N), brel_p.reshape(1, 1), h3,
                    wmlp, bmlp)
    return out


# speedup vs baseline: 14.4514x; 1.3217x over previous
import functools

import jax
import jax.numpy as jnp
from jax import lax
from jax.experimental import pallas as pl
from jax.experimental.pallas import tpu as pltpu
from jax.experimental.pallas import tpu_sc as plsc

_N = 10000
_E = 320000
_H = 128
_OUT = 20
_EPS = 1e-5
_NC = 2
_NS = 16
_NW = _NC * _NS
_EPW = _E // _NW
_CH = 80
_NCHUNK = _EPW // _CH
_NP = 10240
_RPT = _NP // _NS


def _row_segsum_body(h_hbm, src_hbm, dst_hbm, z_hbm, out_hbm,
                     src_v, db0, db1, db2, b0, b1, b2, acc_sh,
                     g0, g1, g2, s0, s1, s2, d0, d1, d2):
    bufs = (b0, b1, b2)
    dbs = (db0, db1, db2)
    gsem = (g0, g1, g2)
    ssem = (s0, s1, s2)
    dsem = (d0, d1, d2)
    c = lax.axis_index("c")
    sub = lax.axis_index("s")
    wid = sub * _NC + c
    row0 = sub * _RPT
    pltpu.sync_copy(z_hbm, acc_sh.at[pl.ds(row0, _RPT)])
    pltpu.sync_copy(src_hbm.at[wid], src_v)
    plsc.subcore_barrier()

    def load(j, k):
        off = pl.multiple_of(wid * _EPW + j * _CH, 8)
        pltpu.async_copy(dst_hbm.at[pl.ds(off, _CH)], dbs[k], dsem[k])
        pltpu.async_copy(h_hbm.at[src_v.at[j]], bufs[k], gsem[k])

    def wait_load(j, k):
        off = pl.multiple_of(wid * _EPW + j * _CH, 8)
        pltpu.make_async_copy(h_hbm.at[src_v.at[j]], bufs[k], gsem[k]).wait()
        pltpu.make_async_copy(dst_hbm.at[pl.ds(off, _CH)], dbs[k], dsem[k]).wait()

    def scatter(k):
        pltpu.async_copy(bufs[k], acc_sh.at[dbs[k]], ssem[k], add=True)

    def wait_scatter(k):
        pltpu.make_async_copy(bufs[k], acc_sh.at[dbs[k]], ssem[k]).wait()

    load(0, 0)
    load(1, 1)
    wait_load(0, 0)
    scatter(0)
    load(2, 2)

    def body(t, carry):
        j_base = 3 * t + 1
        for i in range(3):
            j = j_base + i
            k = (1 + i) % 3
            wait_load(j, k)
            scatter(k)
            nxt = j + 2

            @pl.when(nxt < _NCHUNK)
            def _():
                wait_scatter(i)
                load(nxt, i)

        return carry

    lax.fori_loop(0, (_NCHUNK - 2) // 3, body, 0)
    last = _NCHUNK - 1
    wait_load(last, last % 3)
    scatter(last % 3)
    for k in range(3):
        wait_scatter(k)
    plsc.subcore_barrier()
    pltpu.sync_copy(acc_sh.at[pl.ds(row0, _RPT)], out_hbm.at[c, pl.ds(row0, _RPT)])


def _sc_row_segsum(h, src2, dst2, ztile):
    mesh = plsc.VectorSubcoreMesh(core_axis_name="c", subcore_axis_name="s")
    kern = pl.kernel(
        _row_segsum_body,
        out_type=jax.ShapeDtypeStruct((_NC, _NP, _H), jnp.float32),
        mesh=mesh,
        scratch_types=[
            pltpu.VMEM((_NCHUNK, _CH), jnp.int32),
            pltpu.VMEM((_CH,), jnp.int32),
            pltpu.VMEM((_CH,), jnp.int32),
            pltpu.VMEM((_CH,), jnp.int32),
            pltpu.VMEM((_CH, _H), jnp.float32),
            pltpu.VMEM((_CH, _H), jnp.float32),
            pltpu.VMEM((_CH, _H), jnp.float32),
            pltpu.VMEM_SHARED((_NP, _H), jnp.float32),
        ] + [pltpu.SemaphoreType.DMA] * 9,
    )
    return kern(h, src2, dst2, ztile)


def _scal_segsum_body(srel_hbm, src_hbm, dst_hbm, z_hbm, out_hbm,
                      srel_v, src_v, dst_v, acc_v):
    c = lax.axis_index("c")
    s = lax.axis_index("s")
    wid = s * _NC + c
    pltpu.sync_copy(srel_hbm, srel_v)
    pltpu.sync_copy(z_hbm, acc_v)
    base = wid * _EPW
    pltpu.sync_copy(src_hbm.at[pl.ds(base, _EPW)], src_v)
    pltpu.sync_copy(dst_hbm.at[pl.ds(base, _EPW)], dst_v)

    def body(i, carry):
        off = pl.multiple_of(i * 16, 16)
        idx = src_v[pl.ds(off, 16)]
        vals = plsc.load_gather(srel_v, [idx])
        didx = dst_v[pl.ds(off, 16)]
        plsc.addupdate_scatter(acc_v, [didx], vals)
        return carry

    lax.fori_loop(0, _EPW // 16, body, 0)
    pltpu.sync_copy(acc_v, out_hbm.at[pl.ds(wid * _N, _N)])


def _sc_scal_segsum(srel, src, dst, zvec):
    mesh = plsc.VectorSubcoreMesh(core_axis_name="c", subcore_axis_name="s")
    kern = pl.kernel(
        _scal_segsum_body,
        out_type=jax.ShapeDtypeStruct((_NW * _N,), jnp.float32),
        mesh=mesh,
        compiler_params=pltpu.CompilerParams(needs_layout_passes=False),
        scratch_types=[
            pltpu.VMEM((_N,), jnp.float32),
            pltpu.VMEM((_EPW,), jnp.int32),
            pltpu.VMEM((_EPW,), jnp.int32),
            pltpu.VMEM((_N,), jnp.float32),
        ],
    )
    return kern(srel, src, dst, zvec)


def _pre_body(x_ref, vn_ref, o_ref):
    o_ref[...] = x_ref[...] + vn_ref[...]


def _tc_pre(x, vn0):
    return pl.pallas_call(
        _pre_body,
        out_shape=jax.ShapeDtypeStruct((_N, _H), jnp.float32),
    )(x, vn0)


def _dense_body(p_ref, h_ref, wr_ref, br_ref, ws_ref, g_ref, b_ref,
                vn_ref, o_ref):
    agg = p_ref[0, pl.ds(0, _N), :] + p_ref[1, pl.ds(0, _N), :]
    h = h_ref[...]
    z = (jnp.dot(agg, wr_ref[...], preferred_element_type=jnp.float32)
         + br_ref[...]
         + jnp.dot(h, ws_ref[...], preferred_element_type=jnp.float32))
    mean = jnp.mean(z, axis=0, keepdims=True)
    zc = z - mean
    var = jnp.mean(zc * zc, axis=0, keepdims=True)
    hn = g_ref[...] * zc * lax.rsqrt(var + _EPS) + b_ref[...]
    o_ref[...] = jnp.maximum(hn, 0.0) + vn_ref[...]


def _dense_last_body(p_ref, h_ref, wr_ref, br_ref, ws_ref, g_ref, b_ref,
                     wrp_ref, wsp_ref, o_ref, srel_ref, sroot_ref):
    agg = p_ref[0, pl.ds(0, _N), :] + p_ref[1, pl.ds(0, _N), :]
    h = h_ref[...]
    z = (jnp.dot(agg, wr_ref[...], preferred_element_type=jnp.float32)
         + br_ref[...]
         + jnp.dot(h, ws_ref[...], preferred_element_type=jnp.float32))
    mean = jnp.mean(z, axis=0, keepdims=True)
    zc = z - mean
    var = jnp.mean(zc * zc, axis=0, keepdims=True)
    hn = g_ref[...] * zc * lax.rsqrt(var + _EPS) + b_ref[...]
    h3 = jnp.maximum(hn, 0.0)
    o_ref[...] = h3
    srel_ref[...] = jnp.dot(h3, wrp_ref[...], preferred_element_type=jnp.float32)
    sroot_ref[...] = jnp.dot(h3, wsp_ref[...], preferred_element_type=jnp.float32)


def _tc_dense(p, h, wr, br, ws, g, b, vn_next):
    return pl.pallas_call(
        _dense_body,
        out_shape=jax.ShapeDtypeStruct((_N, _H), jnp.float32),
    )(p, h, wr, br, ws, g, b, vn_next)


def _tc_dense_last(p, h, wr, br, ws, g, b, wrp, wsp):
    return pl.pallas_call(
        _dense_last_body,
        out_shape=(
            jax.ShapeDtypeStruct((_N, _H), jnp.float32),
            jax.ShapeDtypeStruct((_N, 1), jnp.float32),
            jax.ShapeDtypeStruct((_N, 1), jnp.float32),
        ),
    )(p, h, wr, br, ws, g, b, wrp, wsp)


def _final_body(parts_ref, sroot_ref, brp_ref, h3_ref, wmlp_ref, bmlp_ref,
                o_ref):
    score = (jnp.sum(parts_ref[...], axis=0, keepdims=True)
             + sroot_ref[...] + brp_ref[...])
    m = jnp.max(score)
    iota = lax.broadcasted_iota(jnp.int32, (1, _N), 1)
    am = jnp.min(jnp.where(score == m, iota, _N))
    row = h3_ref[pl.ds(am, 1), :]
    hp = row * jnp.tanh(m)
    o_ref[...] = (jnp.dot(hp, wmlp_ref[...], preferred_element_type=jnp.float32)
                  + bmlp_ref[...])


def _tc_final(parts, sroot, brp, h3, wmlp, bmlp):
    return pl.pallas_call(
        _final_body,
        out_shape=jax.ShapeDtypeStruct((1, _OUT), jnp.float32),
    )(parts, sroot, brp, h3, wmlp, bmlp)


def kernel(x, edge_index, edge_attr, batch, vn0, wrel0, brel0, wroot0, gam0,
           bet0, vn1, wrel1, brel1, wroot1, gam1, bet1, vn2, wrel2, brel2,
           wroot2, gam2, bet2, wrel_p, brel_p, wroot_p, wmlp, bmlp):
    del edge_attr, batch
    src = edge_index[0]
    dst = edge_index[1]
    src2 = src.reshape(_NW, _NCHUNK, _CH)
    ztile = jnp.zeros((_RPT, _H), jnp.float32)
    zvec = jnp.zeros((_N,), jnp.float32)

    layers = [
        (wrel0, brel0.reshape(1, _H), wroot0, gam0.reshape(1, _H),
         bet0.reshape(1, _H)),
        (wrel1, brel1.reshape(1, _H), wroot1, gam1.reshape(1, _H),
         bet1.reshape(1, _H)),
        (wrel2, brel2.reshape(1, _H), wroot2, gam2.reshape(1, _H),
         bet2.reshape(1, _H)),
    ]

    h = _tc_pre(x, vn0)
    for i in range(2):
        wr, br, ws, g, b = layers[i]
        p = _sc_row_segsum(h, src2, dst, ztile)
        vn_next = vn1 if i == 0 else vn2
        h = _tc_dense(p, h, wr, br, ws, g, b, vn_next)

    wr, br, ws, g, b = layers[2]
    p = _sc_row_segsum(h, src2, dst, ztile)
    h3, srel, sroot = _tc_dense_last(p, h, wr, br, ws, g, b, wrel_p, wroot_p)

    parts = _sc_scal_segsum(srel.reshape(_N), src, dst, zvec).reshape(_NW, _N)
    out = _tc_final(parts, sroot.reshape(1, _N), brel_p.reshape(1, 1), h3,
                    wmlp, bmlp)
    return out
